# Initial kernel scaffold; baseline (speedup 1.0000x reference)
import functools

import jax
import jax.numpy as jnp
from jax import lax
from jax.experimental import pallas as pl
from jax.experimental.pallas import tpu as pltpu
from jax.experimental.pallas import tpu_sc as plsc

B = 262144
C = 32
RES = 512
HW = RES * RES
PLANE_DIMS = ((0, 1), (0, 2), (1, 2))

NUM_WORKERS = 32
PTS_PER_W = B // NUM_WORKERS
CHUNK = 128
NCHUNKS = PTS_PER_W // CHUNK
NGROUPS = CHUNK // 16


def _sc_body(x0h, x1h, x2h, t0h, t1h, t2h, outh,
             xb, fb, idxb,
             g0, g1, g2, g3, g4, g5, g6, g7, g8, g9, g10, g11,
             outb, sem):
    gbufs = (g0, g1, g2, g3, g4, g5, g6, g7, g8, g9, g10, g11)
    tabs = (t0h, t1h, t2h)
    wid = lax.axis_index("s") * 2 + lax.axis_index("c")
    base = wid * PTS_PER_W

    pltpu.sync_copy(x0h.at[pl.ds(base, PTS_PER_W)], xb.at[0])
    pltpu.sync_copy(x1h.at[pl.ds(base, PTS_PER_W)], xb.at[1])
    pltpu.sync_copy(x2h.at[pl.ds(base, PTS_PER_W)], xb.at[2])

    def chunk_body(j, carry):
        coff = j * CHUNK

        for g in range(NGROUPS):
            off = g * 16
            lo, lop = [], []
            for d in range(3):
                xd = xb[d, pl.ds(coff + off, 16)]
                ix = (xd + 1.0) * 0.5 * float(RES - 1)
                ii = ix.astype(jnp.int32)
                ii = jnp.clip(ii, 0, RES - 1)
                fd = ix - ii.astype(jnp.float32)
                iip = jnp.minimum(ii + 1, RES - 1)
                fb[d, pl.ds(off, 16)] = fd
                lo.append(ii)
                lop.append(iip)
            for p, (d0, d1) in enumerate(PLANE_DIMS):
                r0 = lo[d1] * RES
                r1 = lop[d1] * RES
                idxb[4 * p + 0, pl.ds(off, 16)] = r0 + lo[d0]
                idxb[4 * p + 1, pl.ds(off, 16)] = r0 + lop[d0]
                idxb[4 * p + 2, pl.ds(off, 16)] = r1 + lo[d0]
                idxb[4 * p + 3, pl.ds(off, 16)] = r1 + lop[d0]

        copies = []
        for p in range(3):
            for t in range(4):
                k = 4 * p + t
                copies.append(pltpu.async_copy(tabs[p].at[idxb.at[k]], gbufs[k], sem))
        for cp in copies:
            cp.wait()

        def pt_body(b, carry2):
            f0 = fb[0, b]
            f1 = fb[1, b]
            f2 = fb[2, b]
            wxs = (f0, f0, f1)
            wys = (f1, f2, f2)
            for h in range(2):
                hs = pl.ds(h * 16, 16)
                acc = None
                for p in range(3):
                    wx = wxs[p]
                    wy = wys[p]
                    v00 = gbufs[4 * p + 0][b, hs]
                    v01 = gbufs[4 * p + 1][b, hs]
                    v10 = gbufs[4 * p + 2][b, hs]
                    v11 = gbufs[4 * p + 3][b, hs]
                    top = v00 + wx * (v01 - v00)
                    bot = v10 + wx * (v11 - v10)
                    f = top + wy * (bot - top)
                    acc = f if acc is None else acc * f
                outb[b, hs] = acc
            return carry2

        lax.fori_loop(0, CHUNK, pt_body, 0)

        pltpu.sync_copy(outb, outh.at[pl.ds(base + coff, CHUNK)])
        return carry

    lax.fori_loop(0, NCHUNKS, chunk_body, 0)


_sc_call = functools.partial(
    pl.kernel,
    out_type=jax.ShapeDtypeStruct((B, C), jnp.float32),
    mesh=plsc.VectorSubcoreMesh(core_axis_name="c", subcore_axis_name="s"),
    scratch_types=(
        [pltpu.VMEM((3, PTS_PER_W), jnp.float32),
         pltpu.VMEM((3, CHUNK), jnp.float32),
         pltpu.VMEM((12, CHUNK), jnp.int32)]
        + [pltpu.VMEM((CHUNK, C), jnp.float32) for _ in range(12)]
        + [pltpu.VMEM((CHUNK, C), jnp.float32),
           pltpu.SemaphoreType.DMA]
    ),
)(_sc_body)


def kernel(x, plane0, plane1, plane2):
    x0 = x[:, 0]
    x1 = x[:, 1]
    x2 = x[:, 2]
    t0 = plane0[0].transpose(1, 2, 0).reshape(HW, C)
    t1 = plane1[0].transpose(1, 2, 0).reshape(HW, C)
    t2 = plane2[0].transpose(1, 2, 0).reshape(HW, C)
    return _sc_call(x0, x1, x2, t0, t1, t2)


# speedup vs baseline: 2.5713x; 2.5713x over previous
import functools

import jax
import jax.numpy as jnp
from jax import lax
from jax.experimental import pallas as pl
from jax.experimental.pallas import tpu as pltpu
from jax.experimental.pallas import tpu_sc as plsc

B = 262144
C = 32
RES = 512
HW = RES * RES
PLANE_DIMS = ((0, 1), (0, 2), (1, 2))

NUM_WORKERS = 32
PTS_PER_W = B // NUM_WORKERS
CHUNK = 128
NCHUNKS = PTS_PER_W // CHUNK
NGROUPS = CHUNK // 16


def _sc_body(x0h, x1h, x2h, t0h, t1h, t2h, outh, *scr):
    xbufs = scr[0:3]
    fbufs = scr[3:6]
    ibufs = scr[6:18]
    gbufs = scr[18:30]
    outb = scr[30]
    sem = scr[31]
    xhs = (x0h, x1h, x2h)
    tabs = (t0h, t1h, t2h)
    wid = lax.axis_index("s") * 2 + lax.axis_index("c")
    base = wid * PTS_PER_W

    for d in range(3):
        pltpu.sync_copy(xhs[d].at[pl.ds(base, PTS_PER_W)], xbufs[d])

    def chunk_body(j, carry):
        coff = j * CHUNK

        for g in range(NGROUPS):
            off = g * 16
            gs = pl.ds(off, 16)
            lo, lop = [], []
            for d in range(3):
                xd = xbufs[d][pl.ds(coff + off, 16)]
                ix = (xd + 1.0) * 0.5 * float(RES - 1)
                ii = ix.astype(jnp.int32)
                ii = jnp.clip(ii, 0, RES - 1)
                fd = ix - ii.astype(jnp.float32)
                iip = jnp.minimum(ii + 1, RES - 1)
                fbufs[d][gs] = fd
                lo.append(ii)
                lop.append(iip)
            for p, (d0, d1) in enumerate(PLANE_DIMS):
                r0 = lo[d1] * RES
                r1 = lop[d1] * RES
                ibufs[4 * p + 0][gs] = r0 + lo[d0]
                ibufs[4 * p + 1][gs] = r0 + lop[d0]
                ibufs[4 * p + 2][gs] = r1 + lo[d0]
                ibufs[4 * p + 3][gs] = r1 + lop[d0]

        copies = []
        for p in range(3):
            for t in range(4):
                k = 4 * p + t
                copies.append(pltpu.async_copy(tabs[p].at[ibufs[k]], gbufs[k], sem))
        for cp in copies:
            cp.wait()

        def pt_group(g, carry2):
            gs = pl.ds(g * 16, 16)
            f0v = fbufs[0][gs]
            f1v = fbufs[1][gs]
            f2v = fbufs[2][gs]
            for l in range(16):
                b = g * 16 + l
                wxs = (f0v[l], f0v[l], f1v[l])
                wys = (f1v[l], f2v[l], f2v[l])
                for h in range(2):
                    hs = pl.ds(h * 16, 16)
                    acc = None
                    for p in range(3):
                        wx = wxs[p]
                        wy = wys[p]
                        v00 = gbufs[4 * p + 0][b, hs]
                        v01 = gbufs[4 * p + 1][b, hs]
                        v10 = gbufs[4 * p + 2][b, hs]
                        v11 = gbufs[4 * p + 3][b, hs]
                        top = v00 + wx * (v01 - v00)
                        bot = v10 + wx * (v11 - v10)
                        f = top + wy * (bot - top)
                        acc = f if acc is None else acc * f
                    outb[b, hs] = acc
            return carry2

        lax.fori_loop(0, NGROUPS, pt_group, 0)

        pltpu.sync_copy(outb, outh.at[pl.ds(base + coff, CHUNK)])
        return carry

    lax.fori_loop(0, NCHUNKS, chunk_body, 0)


_sc_call = functools.partial(
    pl.kernel,
    out_type=jax.ShapeDtypeStruct((B, C), jnp.float32),
    mesh=plsc.VectorSubcoreMesh(core_axis_name="c", subcore_axis_name="s"),
    compiler_params=pltpu.CompilerParams(use_tc_tiling_on_sc=False),
    scratch_types=(
        [pltpu.VMEM((PTS_PER_W,), jnp.float32) for _ in range(3)]
        + [pltpu.VMEM((CHUNK,), jnp.float32) for _ in range(3)]
        + [pltpu.VMEM((CHUNK,), jnp.int32) for _ in range(12)]
        + [pltpu.VMEM((CHUNK, C), jnp.float32) for _ in range(12)]
        + [pltpu.VMEM((CHUNK, C), jnp.float32),
           pltpu.SemaphoreType.DMA]
    ),
)(_sc_body)


def kernel(x, plane0, plane1, plane2):
    x0 = x[:, 0]
    x1 = x[:, 1]
    x2 = x[:, 2]
    t0 = plane0[0].transpose(1, 2, 0).reshape(HW, C)
    t1 = plane1[0].transpose(1, 2, 0).reshape(HW, C)
    t2 = plane2[0].transpose(1, 2, 0).reshape(HW, C)
    return _sc_call(x0, x1, x2, t0, t1, t2)


# speedup vs baseline: 3.2808x; 1.2759x over previous
import functools

import jax
import jax.numpy as jnp
from jax import lax
from jax.experimental import pallas as pl
from jax.experimental.pallas import tpu as pltpu
from jax.experimental.pallas import tpu_sc as plsc

B = 262144
C = 32
RES = 512
HW = RES * RES
PLANE_DIMS = ((0, 1), (0, 2), (1, 2))

NUM_WORKERS = 32
PTS_PER_W = B // NUM_WORKERS
CHUNK = 128
NCHUNKS = PTS_PER_W // CHUNK
NGROUPS = CHUNK // 16


def _sc_body(x0h, x1h, x2h, t0h, t1h, t2h, outh, *scr):
    xb = (scr[0:3], scr[3:6])
    fb = (scr[6:9], scr[9:12])
    ib = (scr[12:24], scr[24:36])
    gb = (scr[36:48], scr[48:60])
    ob = scr[60:62]
    xsem, gsem, osem = scr[62:64], scr[64:66], scr[66:68]
    xhs = (x0h, x1h, x2h)
    tabs = (t0h, t1h, t2h)
    wid = lax.axis_index("s") * 2 + lax.axis_index("c")
    base = wid * PTS_PER_W

    def x_fire(jj, s):
        for d in range(3):
            pltpu.make_async_copy(
                xhs[d].at[pl.ds(base + jj * CHUNK, CHUNK)], xb[s][d], xsem[s]
            ).start()

    def x_wait(jj, s):
        for d in range(3):
            pltpu.make_async_copy(
                xhs[d].at[pl.ds(base + jj * CHUNK, CHUNK)], xb[s][d], xsem[s]
            ).wait()

    def prep(s):
        for g in range(NGROUPS):
            gs = pl.ds(g * 16, 16)
            lo, lop = [], []
            for d in range(3):
                xd = xb[s][d][gs]
                ix = (xd + 1.0) * 0.5 * float(RES - 1)
                ii = ix.astype(jnp.int32)
                ii = jnp.clip(ii, 0, RES - 1)
                fd = ix - ii.astype(jnp.float32)
                iip = jnp.minimum(ii + 1, RES - 1)
                fb[s][d][gs] = fd
                lo.append(ii)
                lop.append(iip)
            for p, (d0, d1) in enumerate(PLANE_DIMS):
                r0 = lo[d1] * RES
                r1 = lop[d1] * RES
                ib[s][4 * p + 0][gs] = r0 + lo[d0]
                ib[s][4 * p + 1][gs] = r0 + lop[d0]
                ib[s][4 * p + 2][gs] = r1 + lo[d0]
                ib[s][4 * p + 3][gs] = r1 + lop[d0]

    def gather_fire(s):
        for p in range(3):
            for t in range(4):
                k = 4 * p + t
                pltpu.make_async_copy(
                    tabs[p].at[ib[s][k]], gb[s][k], gsem[s]
                ).start()

    def gather_wait(s):
        for p in range(3):
            for t in range(4):
                k = 4 * p + t
                pltpu.make_async_copy(
                    tabs[p].at[ib[s][k]], gb[s][k], gsem[s]
                ).wait()

    def out_fire(jj, s):
        pltpu.make_async_copy(
            ob[s], outh.at[pl.ds(base + jj * CHUNK, CHUNK)], osem[s]
        ).start()

    def out_wait(jj, s):
        pltpu.make_async_copy(
            ob[s], outh.at[pl.ds(base + jj * CHUNK, CHUNK)], osem[s]
        ).wait()

    def compute(s):
        def pt_group(g, carry):
            gs = pl.ds(g * 16, 16)
            f0v = fb[s][0][gs]
            f1v = fb[s][1][gs]
            f2v = fb[s][2][gs]
            for l in range(16):
                bpt = g * 16 + l
                wxs = (f0v[l], f0v[l], f1v[l])
                wys = (f1v[l], f2v[l], f2v[l])
                for h in range(2):
                    hs = pl.ds(h * 16, 16)
                    acc = None
                    for p in range(3):
                        wx = wxs[p]
                        wy = wys[p]
                        v00 = gb[s][4 * p + 0][bpt, hs]
                        v01 = gb[s][4 * p + 1][bpt, hs]
                        v10 = gb[s][4 * p + 2][bpt, hs]
                        v11 = gb[s][4 * p + 3][bpt, hs]
                        top = v00 + wx * (v01 - v00)
                        bot = v10 + wx * (v11 - v10)
                        f = top + wy * (bot - top)
                        acc = f if acc is None else acc * f
                    ob[s][bpt, hs] = acc
            return carry

        lax.fori_loop(0, NGROUPS, pt_group, 0)

    x_fire(0, 0)
    x_wait(0, 0)
    prep(0)
    gather_fire(0)
    x_fire(1, 1)

    def pair_body(jh, carry):
        for par in range(2):
            jj = 2 * jh + par
            s, s2 = par, 1 - par

            @pl.when(jj + 1 < NCHUNKS)
            def _():
                x_wait(jj + 1, s2)
                prep(s2)
                gather_fire(s2)

            @pl.when(jj + 2 < NCHUNKS)
            def _():
                x_fire(jj + 2, s)

            gather_wait(s)

            @pl.when(jj >= 2)
            def _():
                out_wait(jj - 2, s)

            compute(s)
            out_fire(jj, s)
        return carry

    lax.fori_loop(0, NCHUNKS // 2, pair_body, 0)

    out_wait(NCHUNKS - 2, 0)
    out_wait(NCHUNKS - 1, 1)


_sc_call = functools.partial(
    pl.kernel,
    out_type=jax.ShapeDtypeStruct((B, C), jnp.float32),
    mesh=plsc.VectorSubcoreMesh(core_axis_name="c", subcore_axis_name="s"),
    compiler_params=pltpu.CompilerParams(use_tc_tiling_on_sc=False),
    scratch_types=(
        [pltpu.VMEM((CHUNK,), jnp.float32) for _ in range(6)]
        + [pltpu.VMEM((CHUNK,), jnp.float32) for _ in range(6)]
        + [pltpu.VMEM((CHUNK,), jnp.int32) for _ in range(24)]
        + [pltpu.VMEM((CHUNK, C), jnp.float32) for _ in range(24)]
        + [pltpu.VMEM((CHUNK, C), jnp.float32) for _ in range(2)]
        + [pltpu.SemaphoreType.DMA for _ in range(6)]
    ),
)(_sc_body)


def kernel(x, plane0, plane1, plane2):
    x0 = x[:, 0]
    x1 = x[:, 1]
    x2 = x[:, 2]
    t0 = plane0[0].transpose(1, 2, 0).reshape(HW, C)
    t1 = plane1[0].transpose(1, 2, 0).reshape(HW, C)
    t2 = plane2[0].transpose(1, 2, 0).reshape(HW, C)
    return _sc_call(x0, x1, x2, t0, t1, t2)
